# baseline (speedup 1.0000x reference)
import functools

import jax
import jax.numpy as jnp
from jax.experimental import pallas as pl
from jax.experimental.pallas import tpu as pltpu
from jax.experimental.pallas import tpu_sc as plsc

DIM = 64
N_EMBED = 1024
PIX = 1024
BPS = 4


def _vq_body(z_ref, e_ref, idx_ref):
    emb = e_ref[...]
    e2 = jnp.sum(emb * emb, axis=1, keepdims=True)
    emb_m2 = -2.0 * emb
    for j in range(BPS):
        zb = z_ref[j]
        s_m2 = jax.lax.dot_general(
            emb_m2, zb, (((1,), (0,)), ((), ())),
            preferred_element_type=jnp.float32,
            precision=jax.lax.Precision.DEFAULT)
        z2 = jnp.sum(zb * zb, axis=0, keepdims=True)
        d = (z2 + e2) + s_m2
        idx_ref[j, 0] = jnp.argmin(d, axis=0)


def _make_sc_gather(n_rows):
    info = plsc.get_sparse_core_info()
    nw = info.num_cores * info.num_subcores
    rows_per_w = n_rows // nw
    mesh = plsc.VectorSubcoreMesh(core_axis_name="c", subcore_axis_name="s")

    @functools.partial(
        pl.kernel, mesh=mesh,
        out_type=jax.ShapeDtypeStruct((n_rows, 2 * DIM), jnp.float32),
        scratch_types=[
            pltpu.VMEM((rows_per_w,), jnp.int32),
            pltpu.VMEM((rows_per_w, 2 * DIM), jnp.float32),
            pltpu.SemaphoreType.DMA,
        ],
    )
    def gather_k(table_hbm, idx_hbm, out_hbm, idx_v, rows_v, sem):
        wid = jax.lax.axis_index("s") * info.num_cores + jax.lax.axis_index("c")
        base = wid * rows_per_w
        pltpu.sync_copy(idx_hbm.at[pl.ds(base, rows_per_w)], idx_v)
        pltpu.async_copy(table_hbm.at[idx_v], rows_v, sem).wait()
        pltpu.sync_copy(rows_v, out_hbm.at[pl.ds(base, rows_per_w)])

    return gather_k


def kernel(z, embedding):
    b = z.shape[0]
    n_rows = b * PIX
    z3 = z.reshape(b, DIM, PIX)
    idx = pl.pallas_call(
        _vq_body,
        grid=(b // BPS,),
        in_specs=[
            pl.BlockSpec((BPS, DIM, PIX), lambda i: (i, 0, 0)),
            pl.BlockSpec((N_EMBED, DIM), lambda i: (0, 0)),
        ],
        out_specs=pl.BlockSpec((BPS, 1, PIX), lambda i: (i, 0, 0)),
        out_shape=jax.ShapeDtypeStruct((b, 1, PIX), jnp.int32),
    )(z3, embedding)
    idx_flat = idx.reshape(n_rows)
    table = jnp.pad(embedding, ((0, 0), (0, DIM)))
    zq_rows = _make_sc_gather(n_rows)(table, idx_flat)
    zq = zq_rows[:, :DIM].reshape(b, 32, 32, DIM).transpose(0, 3, 1, 2)
    return zq, idx_flat


# speedup vs baseline: 1.6419x; 1.6419x over previous
import jax
import jax.numpy as jnp
from jax.experimental import pallas as pl

DIM = 64
N_EMBED = 1024
PIX = 1024
BPS = 8


def _vq_body(z_ref, e_ref, zq_ref, idx_ref):
    emb = e_ref[...]
    e2 = jnp.sum(emb * emb, axis=1, keepdims=True)
    emb_m2 = -2.0 * emb
    for j in range(BPS):
        zb = z_ref[j]
        s_m2 = jax.lax.dot_general(
            emb_m2, zb, (((1,), (0,)), ((), ())),
            preferred_element_type=jnp.float32,
            precision=jax.lax.Precision.DEFAULT)
        z2 = jnp.sum(zb * zb, axis=0, keepdims=True)
        d = (z2 + e2) + s_m2
        idx = jnp.argmin(d, axis=0)
        idx_ref[j, 0] = idx
        onehot = (jax.lax.broadcasted_iota(jnp.int32, (N_EMBED, PIX), 0)
                  == idx[None, :]).astype(jnp.bfloat16)
        zq_ref[j] = jax.lax.dot_general(
            emb, onehot, (((0,), (0,)), ((), ())),
            preferred_element_type=jnp.float32,
            precision=jax.lax.Precision.DEFAULT)


def kernel(z, embedding):
    b = z.shape[0]
    z3 = z.reshape(b, DIM, PIX)
    zq, idx = pl.pallas_call(
        _vq_body,
        grid=(b // BPS,),
        in_specs=[
            pl.BlockSpec((BPS, DIM, PIX), lambda i: (i, 0, 0)),
            pl.BlockSpec((N_EMBED, DIM), lambda i: (0, 0)),
        ],
        out_specs=[
            pl.BlockSpec((BPS, DIM, PIX), lambda i: (i, 0, 0)),
            pl.BlockSpec((BPS, 1, PIX), lambda i: (i, 0, 0)),
        ],
        out_shape=[
            jax.ShapeDtypeStruct((b, DIM, PIX), jnp.float32),
            jax.ShapeDtypeStruct((b, 1, PIX), jnp.int32),
        ],
    )(z3, embedding)
    return zq.reshape(z.shape), idx.reshape(b * PIX)
